# baseline (speedup 1.0000x reference)
import functools

import jax
import jax.numpy as jnp
from jax import lax
from jax.experimental import pallas as pl
from jax.experimental.pallas import tpu as pltpu
from jax.experimental.pallas import tpu_sc as plsc

NUM_EMBEDDINGS = 1000000
D = 64
BATCH = 4096
SEQ_LEN = 200
B = BATCH * SEQ_LEN

NC = 2
NS = 16
NW = NC * NS

BLOCK = 256
BLOCKS_PER_T = BATCH // BLOCK
NBLOCKS = SEQ_LEN * BLOCKS_PER_T
NBLK = NBLOCKS // NW
NPAIR = NBLK // 2


def _make_kernel():
    mesh = plsc.VectorSubcoreMesh(core_axis_name="c", subcore_axis_name="s")

    @functools.partial(
        pl.kernel,
        out_type=jax.ShapeDtypeStruct((SEQ_LEN, 8, BATCH // 128, 8, 128),
                                      jnp.float32),
        mesh=mesh,
        scratch_types=[
            pltpu.VMEM((BLOCK,), jnp.int32),
            pltpu.VMEM((BLOCK,), jnp.int32),
            pltpu.VMEM((BLOCK, D), jnp.float32),
            pltpu.VMEM((BLOCK, D), jnp.float32),
            pltpu.VMEM((8, 2, 8, 128), jnp.float32),
            pltpu.VMEM((8, 2, 8, 128), jnp.float32),
            pltpu.SemaphoreType.DMA,
            pltpu.SemaphoreType.DMA,
            pltpu.SemaphoreType.DMA,
            pltpu.SemaphoreType.DMA,
            pltpu.SemaphoreType.DMA,
            pltpu.SemaphoreType.DMA,
        ],
        compiler_params=pltpu.CompilerParams(use_tc_tiling_on_sc=False, needs_layout_passes=False),
    )
    def emb_kernel(idxT_hbm, table_hbm, out_hbm,
                   idx0, idx1, grows0, grows1, gt0, gt1,
                   isem0, isem1, gsem0, gsem1, wsem0, wsem1):
        wid = lax.axis_index("s") * NC + lax.axis_index("c")
        base = wid * NBLK
        idx_v = (idx0, idx1)
        grows = (grows0, grows1)
        gt = (gt0, gt1)
        isem = (isem0, isem1)
        gsem = (gsem0, gsem1)
        wsem = (wsem0, wsem1)

        iota = lax.iota(jnp.int32, 16)

        def idx_copy(g, b):
            t = g // BLOCKS_PER_T
            bk = g % BLOCKS_PER_T
            return pltpu.make_async_copy(
                idxT_hbm.at[t, pl.ds(bk * BLOCK, BLOCK)], idx_v[b], isem[b])

        def gather(b):
            return pltpu.make_async_copy(
                table_hbm.at[idx_v[b]], grows[b], gsem[b])

        def writeback(g, b):
            t = g // BLOCKS_PER_T
            bk = g % BLOCKS_PER_T
            return pltpu.make_async_copy(
                gt[b], out_hbm.at[t, :, pl.ds(2 * bk, 2), :, :], wsem[b])

        def transpose_block(src, dst):
            def body(r, carry):
                dr = r // 16
                rem = r - dr * 16
                bc2 = rem // 8
                ds = rem - bc2 * 8
                d = dr * 8 + ds
                cols = jnp.zeros((16,), jnp.int32) + d
                rb = bc2 * 128
                for j in range(8):
                    rows = iota + (rb + 16 * j)
                    vals = plsc.load_gather(src, [rows, cols])
                    dst[dr, bc2, ds, pl.ds(16 * j, 16)] = vals
                return carry

            lax.fori_loop(0, 128, body, None)

        idx_copy(base + 0, 0).start()
        idx_copy(base + 1, 1).start()

        def pair(p, carry):
            for b in range(2):
                k = 2 * p + b
                g = base + k
                o = b ^ 1

                @pl.when(k >= 2)
                def _wait_prev_wb():
                    writeback(g - 2, b).wait()

                idx_copy(g, b).wait()
                gather(b).start()

                @pl.when(k >= 1)
                def _drain_and_flush_prev():
                    gather(o).wait()

                    @pl.when(k + 1 < NBLK)
                    def _prefetch_idx():
                        idx_copy(g + 1, o).start()

                    transpose_block(grows[o], gt[o])
                    writeback(g - 1, o).start()

            return carry

        lax.fori_loop(0, NPAIR, pair, None)

        last = base + NBLK - 1
        gather(1).wait()
        transpose_block(grows[1], gt[1])
        writeback(last, 1).start()
        writeback(last - 1, 0).wait()
        writeback(last, 1).wait()

    return emb_kernel


_emb_kernel = _make_kernel()


def kernel(input_, W):
    idxT = input_.T.astype(jnp.int32)
    ot5 = _emb_kernel(idxT, W)
    return ot5.transpose(2, 4, 0, 1, 3).reshape(BATCH, SEQ_LEN, D)


# speedup vs baseline: 1.2912x; 1.2912x over previous
import functools

import jax
import jax.numpy as jnp
from jax import lax
from jax.experimental import pallas as pl
from jax.experimental.pallas import tpu as pltpu
from jax.experimental.pallas import tpu_sc as plsc

NUM_EMBEDDINGS = 1000000
D = 64
BATCH = 4096
SEQ_LEN = 200
B = BATCH * SEQ_LEN

NC = 2
NS = 16
NW = NC * NS

Q = 80

VCT = NUM_EMBEDDINGS // 128
VREM = NUM_EMBEDDINGS - VCT * 128
AMAX = (VCT + NW - 1) // NW

BLOCK = 256
BLOCKS_PER_T = BATCH // BLOCK
NBLOCKS = SEQ_LEN * BLOCKS_PER_T
NBLK = NBLOCKS // NW
NPAIR = NBLK // 2


def _make_phase_a():
    mesh = plsc.VectorSubcoreMesh(core_axis_name="c", subcore_axis_name="s")

    @functools.partial(
        pl.kernel,
        out_type=jax.ShapeDtypeStruct((NUM_EMBEDDINGS * Q,), jnp.float32),
        mesh=mesh,
        scratch_types=[
            pltpu.VMEM((D, 128), jnp.float32),
            pltpu.VMEM((D, 128), jnp.float32),
            pltpu.VMEM((128 * Q,), jnp.float32),
            pltpu.VMEM((128 * Q,), jnp.float32),
            pltpu.VMEM((VREM * D,), jnp.float32),
            pltpu.SemaphoreType.DMA,
            pltpu.SemaphoreType.DMA,
            pltpu.SemaphoreType.DMA,
            pltpu.SemaphoreType.DMA,
        ],
        compiler_params=pltpu.CompilerParams(use_tc_tiling_on_sc=True,
                                             needs_layout_passes=False),
    )
    def phase_a(wt_hbm, wtail_hbm, out_hbm, tin0, tin1, tout0, tout1,
                tin_tail, isem0, isem1, osem0, osem1):
        wid = lax.axis_index("s") * NC + lax.axis_index("c")
        tin = (tin0, tin1)
        tout = (tout0, tout1)
        isem = (isem0, isem1)
        osem = (osem0, osem1)

        iota = lax.iota(jnp.int32, 16)
        iq = iota * Q

        def vc_of(r):
            return r * NW + wid

        def in_copies(r, b):
            vc = vc_of(r)
            return [
                pltpu.make_async_copy(
                    wt_hbm.at[pl.ds(dr * 8, 8), pl.ds(vc * 128, 128)],
                    tin[b].at[pl.ds(dr * 8, 8), :],
                    isem[b])
                for dr in range(8)
            ]

        def out_copy(r, b):
            vc = vc_of(r)
            return pltpu.make_async_copy(
                tout[b], out_hbm.at[pl.ds(vc * 128 * Q, 128 * Q)], osem[b])

        def transpose_tile(src, dst, nv16):
            def body(d, carry):
                for i in range(nv16):
                    vals = src[d, pl.ds(16 * i, 16)]
                    plsc.store_scatter(dst, [iq + (16 * i * Q + d)], vals)
                return carry

            lax.fori_loop(0, D, body, None, unroll=4)

        def valid(r):
            return vc_of(r) < VCT

        def process(r, b):
            @pl.when(valid(r + 1))
            def _start_next():
                for c in in_copies(r + 1, 1 - b):
                    c.start()

            @pl.when(valid(r))
            def _process():
                @pl.when(r >= 2)
                def _():
                    out_copy(r - 2, b).wait()
                for c in in_copies(r, b):
                    c.wait()
                transpose_tile(tin[b], tout[b], 8)
                out_copy(r, b).start()

        @pl.when(valid(0))
        def _():
            for c in in_copies(0, 0):
                c.start()

        def pair_body(p, carry):
            process(2 * p, 0)
            process(2 * p + 1, 1)
            return carry

        lax.fori_loop(0, (AMAX + 1) // 2, pair_body, None)

        nv = (VCT - wid + NW - 1) // NW
        r1 = nv - 1
        r2 = nv - 2

        def _drain(rbuf0, rbuf1):
            out_copy(rbuf0, 0).wait()
            out_copy(rbuf1, 1).wait()

        lax.cond(lax.rem(r1, 2) == 0,
                 lambda: _drain(r1, r2),
                 lambda: _drain(r2, r1))

        @pl.when(wid == NW - 1)
        def _tail():
            pltpu.async_copy(wtail_hbm, tin_tail, isem0)
            pltpu.make_async_copy(wtail_hbm, tin_tail, isem0).wait()

            def tbody(d, carry):
                for i in range(VREM // 16):
                    vals = tin_tail[pl.ds(d * VREM + 16 * i, 16)]
                    plsc.store_scatter(tout0, [iq + (16 * i * Q + d)], vals)
                return carry

            lax.fori_loop(0, D, tbody, None, unroll=4)
            pltpu.async_copy(
                tout0.at[pl.ds(0, VREM * Q)],
                out_hbm.at[pl.ds(VCT * 128 * Q, VREM * Q)],
                osem0)
            pltpu.make_async_copy(
                tout0.at[pl.ds(0, VREM * Q)],
                out_hbm.at[pl.ds(VCT * 128 * Q, VREM * Q)],
                osem0).wait()

    return phase_a


def _make_phase_b():
    mesh = plsc.VectorSubcoreMesh(core_axis_name="c", subcore_axis_name="s")

    @functools.partial(
        pl.kernel,
        out_type=jax.ShapeDtypeStruct((SEQ_LEN, 8, BATCH // 128, 8, 128),
                                      jnp.float32),
        mesh=mesh,
        scratch_types=[
            pltpu.VMEM((BLOCK,), jnp.int32),
            pltpu.VMEM((BLOCK,), jnp.int32),
            pltpu.VMEM((BLOCK, Q), jnp.float32),
            pltpu.VMEM((BLOCK, Q), jnp.float32),
            pltpu.VMEM((8, 2, 8, 128), jnp.float32),
            pltpu.VMEM((8, 2, 8, 128), jnp.float32),
            pltpu.VMEM((BLOCK * 65,), jnp.float32),
            pltpu.SemaphoreType.DMA,
            pltpu.SemaphoreType.DMA,
            pltpu.SemaphoreType.DMA,
            pltpu.SemaphoreType.DMA,
            pltpu.SemaphoreType.DMA,
            pltpu.SemaphoreType.DMA,
        ],
        compiler_params=pltpu.CompilerParams(use_tc_tiling_on_sc=False,
                                             needs_layout_passes=False),
    )
    def phase_b(idxT_hbm, table_hbm, out_hbm,
                idx0, idx1, grows0, grows1, gt0, gt1, gp,
                isem0, isem1, gsem0, gsem1, wsem0, wsem1):
        wid = lax.axis_index("s") * NC + lax.axis_index("c")
        base = wid * NBLK
        idx_v = (idx0, idx1)
        grows = (grows0, grows1)
        gt = (gt0, gt1)
        isem = (isem0, isem1)
        gsem = (gsem0, gsem1)
        wsem = (wsem0, wsem1)

        iota = lax.iota(jnp.int32, 16)

        def idx_copy(g, b):
            t = g // BLOCKS_PER_T
            bk = g % BLOCKS_PER_T
            return pltpu.make_async_copy(
                idxT_hbm.at[t, pl.ds(bk * BLOCK, BLOCK)], idx_v[b], isem[b])

        def gather(b):
            return pltpu.make_async_copy(
                table_hbm.at[idx_v[b]], grows[b], gsem[b])

        def writeback(g, b):
            t = g // BLOCKS_PER_T
            bk = g % BLOCKS_PER_T
            return pltpu.make_async_copy(
                gt[b], out_hbm.at[t, :, pl.ds(2 * bk, 2), :, :], wsem[b])

        i65 = iota * 65

        def transpose_block(src, dst):
            def rbody(l, carry):
                o = l * 65
                for c in range(4):
                    gp[pl.ds(o + 16 * c, 16)] = src[l, pl.ds(16 * c, 16)]
                return carry

            lax.fori_loop(0, BLOCK, rbody, None, unroll=4)

            def body(r, carry):
                dr = r // 16
                rem = r - dr * 16
                bc2 = rem // 8
                ds = rem - bc2 * 8
                d = dr * 8 + ds
                rb = bc2 * 128
                for j in range(8):
                    pos = i65 + ((rb + 16 * j) * 65 + d)
                    vals = plsc.load_gather(gp, [pos])
                    dst[dr, bc2, ds, pl.ds(16 * j, 16)] = vals
                return carry

            lax.fori_loop(0, 128, body, None, unroll=2)

        idx_copy(base + 0, 0).start()
        idx_copy(base + 1, 1).start()

        def pair(p, carry):
            for b in range(2):
                k = 2 * p + b
                g = base + k
                o = b ^ 1

                @pl.when(k >= 2)
                def _wait_prev_wb():
                    writeback(g - 2, b).wait()

                idx_copy(g, b).wait()
                gather(b).start()

                @pl.when(k >= 1)
                def _drain_and_flush_prev():
                    gather(o).wait()

                    @pl.when(k + 1 < NBLK)
                    def _prefetch_idx():
                        idx_copy(g + 1, o).start()

                    transpose_block(grows[o], gt[o])
                    writeback(g - 1, o).start()

            return carry

        lax.fori_loop(0, NPAIR, pair, None)

        last = base + NBLK - 1
        gather(1).wait()
        transpose_block(grows[1], gt[1])
        writeback(last, 1).start()
        writeback(last - 1, 0).wait()
        writeback(last, 1).wait()

    return phase_b


_phase_a = _make_phase_a()
_phase_b = _make_phase_b()


def kernel(input_, W):
    idxT = input_.T.astype(jnp.int32)
    wtail = W[VCT * 128:, :].T.reshape(VREM * D)
    wr = _phase_a(W.T, wtail)
    ot5 = _phase_b(idxT, wr.reshape(NUM_EMBEDDINGS, Q))
    return ot5.transpose(2, 4, 0, 1, 3).reshape(BATCH, SEQ_LEN, D)
